# Initial kernel scaffold; baseline (speedup 1.0000x reference)
import functools

import jax
import jax.numpy as jnp
from jax import lax
from jax.experimental import pallas as pl
from jax.experimental.pallas import tpu as pltpu
from jax.experimental.pallas import tpu_sc as plsc

N_NODES = 10000
N_PAD = 10240
N_EDGES = 320000
D_IN = 128
D_H = 256
D_HALF = 128
D_OUT = 128
N_LAYERS = 4
N_GROUPS = 16

NC = 2
NS = 16
K = 80
EDGES_PER_SUB = N_EDGES // NS
CHUNKS = EDGES_PER_SUB // K
ROWS_PER_SUB = N_PAD // NS
LANES = 16

RB = 1024
GRID = N_PAD // RB


def _build_edge_kernel(with_deg):
  mesh = plsc.VectorSubcoreMesh(core_axis_name="c", subcore_axis_name="s")
  out_type = [
      jax.ShapeDtypeStruct((N_PAD, D_HALF), jnp.float32),
      jax.ShapeDtypeStruct((N_PAD, D_HALF), jnp.float32),
  ]
  scratch = [
      pltpu.VMEM((K,), jnp.int32),
      pltpu.VMEM((K,), jnp.int32),
      pltpu.VMEM((K, D_HALF), jnp.float32),
      pltpu.VMEM((K, D_HALF), jnp.float32),
      pltpu.VMEM((K, D_HALF), jnp.float32),
      pltpu.VMEM_SHARED((N_PAD, D_HALF), jnp.float32),
      pltpu.SemaphoreType.DMA,
      pltpu.SemaphoreType.DMA,
  ]
  if with_deg:
    out_type.append(jax.ShapeDtypeStruct((N_PAD, LANES), jnp.float32))
    scratch += [
        pltpu.VMEM((K, LANES), jnp.float32),
        pltpu.VMEM((ROWS_PER_SUB, LANES), jnp.float32),
        pltpu.VMEM_SHARED((N_PAD, LANES), jnp.float32),
    ]

  def body(row_hbm, col_hbm, a0_hbm, a1_hbm, b0_hbm, b1_hbm, *rest):
    if with_deg:
      (u0_hbm, u1_hbm, deg_hbm, row_v, col_v, a_v, b_v, z_v, u_sh,
       sem_a, sem_b, ones_v, zdeg_v, deg_sh) = rest
    else:
      (u0_hbm, u1_hbm, row_v, col_v, a_v, b_v, z_v, u_sh,
       sem_a, sem_b) = rest

    c = lax.axis_index("c")
    s = lax.axis_index("s")

    zero16 = jnp.zeros((LANES,), jnp.float32)

    def zrow(i, carry):
      for f in range(D_HALF // LANES):
        z_v[i, pl.ds(f * LANES, LANES)] = zero16
      return carry

    lax.fori_loop(0, K, zrow, 0)

    for j in range(ROWS_PER_SUB // K):
      pltpu.sync_copy(z_v, u_sh.at[pl.ds(s * ROWS_PER_SUB + j * K, K)])

    if with_deg:
      one16 = jnp.ones((LANES,), jnp.float32)

      def orow(i, carry):
        ones_v[i, pl.ds(0, LANES)] = one16
        return carry

      lax.fori_loop(0, K, orow, 0)

      def zdrow(i, carry):
        zdeg_v[i, pl.ds(0, LANES)] = zero16
        return carry

      lax.fori_loop(0, ROWS_PER_SUB, zdrow, 0)

      @pl.when(c == 0)
      def _():
        pltpu.sync_copy(zdeg_v, deg_sh.at[pl.ds(s * ROWS_PER_SUB,
                                                ROWS_PER_SUB)])

    plsc.subcore_barrier()

    base = s * EDGES_PER_SUB

    def chunk(t, carry):
      off = base + t * K
      pltpu.sync_copy(row_hbm.at[pl.ds(off, K)], row_v)
      pltpu.sync_copy(col_hbm.at[pl.ds(off, K)], col_v)

      @pl.when(c == 0)
      def _():
        cp1 = pltpu.async_copy(a0_hbm.at[row_v], a_v, sem_a)
        cp2 = pltpu.async_copy(b0_hbm.at[col_v], b_v, sem_b)
        cp1.wait()
        cp2.wait()

      @pl.when(c == 1)
      def _():
        cp1 = pltpu.async_copy(a1_hbm.at[row_v], a_v, sem_a)
        cp2 = pltpu.async_copy(b1_hbm.at[col_v], b_v, sem_b)
        cp1.wait()
        cp2.wait()

      def rowfn(i, rc):
        for f in range(D_HALF // LANES):
          sl = pl.ds(f * LANES, LANES)
          u = a_v[i, sl] + b_v[i, sl]
          a_v[i, sl] = u / (1.0 + jnp.exp(-u))
        return rc

      lax.fori_loop(0, K, rowfn, 0)

      pltpu.sync_copy(a_v, u_sh.at[row_v], add=True)
      if with_deg:
        @pl.when(c == 0)
        def _():
          pltpu.sync_copy(ones_v, deg_sh.at[row_v], add=True)
      return carry

    lax.fori_loop(0, CHUNKS, chunk, 0)

    plsc.subcore_barrier()

    rsl = pl.ds(s * ROWS_PER_SUB, ROWS_PER_SUB)

    @pl.when(c == 0)
    def _():
      pltpu.sync_copy(u_sh.at[rsl], u0_hbm.at[rsl])

    @pl.when(c == 1)
    def _():
      pltpu.sync_copy(u_sh.at[rsl], u1_hbm.at[rsl])

    if with_deg:
      @pl.when(c == 0)
      def _():
        pltpu.sync_copy(deg_sh.at[rsl], deg_hbm.at[rsl])

  return pl.kernel(body, out_type=out_type, mesh=mesh,
                   scratch_types=scratch)


_edge_kernel_deg = _build_edge_kernel(True)
_edge_kernel = _build_edge_kernel(False)


def _dot(a, b):
  return jnp.dot(a, b, preferred_element_type=jnp.float32)


def _tc1_body(x_ref, win_ref, bin_ref, w1a_ref, w1b_ref, bs1_ref,
              h_ref, a0_ref, a1_ref, b0_ref, b1_ref):
  hb = _dot(x_ref[...], win_ref[...]) + bin_ref[...]
  h_ref[...] = hb
  ab = _dot(hb, w1a_ref[...]) + bs1_ref[...]
  a0_ref[...] = ab[:, :D_HALF]
  a1_ref[...] = ab[:, D_HALF:]
  bb = _dot(hb, w1b_ref[...])
  b0_ref[...] = bb[:, :D_HALF]
  b1_ref[...] = bb[:, D_HALF:]


def _agg(h_ref, u0_ref, u1_ref, deg_ref, w2t_ref, w2b_ref, bs2_ref):
  agg = _dot(u0_ref[...], w2t_ref[...]) + _dot(u1_ref[...], w2b_ref[...])
  agg = agg + deg_ref[...][:, 0:1] * bs2_ref[...]
  return h_ref[...] + agg


def _tc2_body(h_ref, u0_ref, u1_ref, deg_ref, w2t_ref, w2b_ref, bs2_ref,
              w1a_ref, w1b_ref, bs1_ref,
              hn_ref, a0_ref, a1_ref, b0_ref, b1_ref):
  hb = _agg(h_ref, u0_ref, u1_ref, deg_ref, w2t_ref, w2b_ref, bs2_ref)
  hn_ref[...] = hb
  ab = _dot(hb, w1a_ref[...]) + bs1_ref[...]
  a0_ref[...] = ab[:, :D_HALF]
  a1_ref[...] = ab[:, D_HALF:]
  bb = _dot(hb, w1b_ref[...])
  b0_ref[...] = bb[:, :D_HALF]
  b1_ref[...] = bb[:, D_HALF:]


def _tc3_body(h_ref, u0_ref, u1_ref, deg_ref, w2t_ref, w2b_ref, bs2_ref,
              wout_ref, bout_ref, bcol_ref,
              hout_ref, pooled_ref, sums_ref, cnts_ref):
  pid = pl.program_id(0)
  hb = _agg(h_ref, u0_ref, u1_ref, deg_ref, w2t_ref, w2b_ref, bs2_ref)
  ho = _dot(hb, wout_ref[...]) + bout_ref[...]
  hout_ref[...] = ho

  bf = bcol_ref[...][:, 0:1]
  gids = lax.broadcasted_iota(jnp.int32, (1, N_GROUPS), 1)
  maskT = (bf == gids).astype(jnp.float32)
  dn = (((0,), (0,)), ((), ()))
  psums = lax.dot_general(maskT, ho, dn,
                          preferred_element_type=jnp.float32)
  pcnts = lax.dot_general(maskT, jnp.ones((RB, D_OUT), jnp.float32), dn,
                          preferred_element_type=jnp.float32)

  @pl.when(pid == 0)
  def _():
    sums_ref[...] = psums
    cnts_ref[...] = pcnts

  @pl.when(pid > 0)
  def _():
    sums_ref[...] += psums
    cnts_ref[...] += pcnts

  @pl.when(pid == GRID - 1)
  def _():
    cnt = cnts_ref[...]
    pooled_ref[...] = sums_ref[...] / cnt / jnp.sqrt(cnt)


def _row_spec(width):
  return pl.BlockSpec((RB, width), lambda i: (i, 0))


def _full_spec(shape):
  return pl.BlockSpec(shape, lambda i: tuple(0 for _ in shape))


_tc1 = pl.pallas_call(
    _tc1_body,
    grid=(GRID,),
    in_specs=[
        _row_spec(D_IN),
        _full_spec((D_IN, D_H)), _full_spec((1, D_H)),
        _full_spec((D_H, D_H)), _full_spec((D_H, D_H)), _full_spec((1, D_H)),
    ],
    out_specs=[
        _row_spec(D_H),
        _row_spec(D_HALF), _row_spec(D_HALF),
        _row_spec(D_HALF), _row_spec(D_HALF),
    ],
    out_shape=[
        jax.ShapeDtypeStruct((N_PAD, D_H), jnp.float32),
        jax.ShapeDtypeStruct((N_PAD, D_HALF), jnp.float32),
        jax.ShapeDtypeStruct((N_PAD, D_HALF), jnp.float32),
        jax.ShapeDtypeStruct((N_PAD, D_HALF), jnp.float32),
        jax.ShapeDtypeStruct((N_PAD, D_HALF), jnp.float32),
    ],
)

_tc2 = pl.pallas_call(
    _tc2_body,
    grid=(GRID,),
    in_specs=[
        _row_spec(D_H),
        _row_spec(D_HALF), _row_spec(D_HALF),
        _row_spec(LANES),
        _full_spec((D_HALF, D_H)), _full_spec((D_HALF, D_H)),
        _full_spec((1, D_H)),
        _full_spec((D_H, D_H)), _full_spec((D_H, D_H)), _full_spec((1, D_H)),
    ],
    out_specs=[
        _row_spec(D_H),
        _row_spec(D_HALF), _row_spec(D_HALF),
        _row_spec(D_HALF), _row_spec(D_HALF),
    ],
    out_shape=[
        jax.ShapeDtypeStruct((N_PAD, D_H), jnp.float32),
        jax.ShapeDtypeStruct((N_PAD, D_HALF), jnp.float32),
        jax.ShapeDtypeStruct((N_PAD, D_HALF), jnp.float32),
        jax.ShapeDtypeStruct((N_PAD, D_HALF), jnp.float32),
        jax.ShapeDtypeStruct((N_PAD, D_HALF), jnp.float32),
    ],
)

_tc3 = pl.pallas_call(
    _tc3_body,
    grid=(GRID,),
    in_specs=[
        _row_spec(D_H),
        _row_spec(D_HALF), _row_spec(D_HALF),
        _row_spec(LANES),
        _full_spec((D_HALF, D_H)), _full_spec((D_HALF, D_H)),
        _full_spec((1, D_H)),
        _full_spec((D_H, D_OUT)), _full_spec((1, D_OUT)),
        _row_spec(8),
    ],
    out_specs=[
        _row_spec(D_OUT),
        _full_spec((N_GROUPS, D_OUT)),
    ],
    out_shape=[
        jax.ShapeDtypeStruct((N_PAD, D_OUT), jnp.float32),
        jax.ShapeDtypeStruct((N_GROUPS, D_OUT), jnp.float32),
    ],
    scratch_shapes=[
        pltpu.VMEM((N_GROUPS, D_OUT), jnp.float32),
        pltpu.VMEM((N_GROUPS, D_OUT), jnp.float32),
    ],
)


@jax.jit
def kernel(x, edge_index, batch, W_in, b_in, Ws1, bs1, Ws2, bs2, W_out,
           b_out):
  row = edge_index[0]
  col = edge_index[1]
  x_pad = jnp.zeros((N_PAD, D_IN), jnp.float32).at[:N_NODES].set(x)
  bcol = jnp.full((N_PAD,), N_GROUPS, jnp.int32).at[:N_NODES].set(batch)
  bcol = jnp.broadcast_to(bcol[:, None], (N_PAD, 8))

  h, A0, A1, B0, B1 = _tc1(x_pad, W_in, b_in[None], Ws1[0, :D_H],
                           Ws1[0, D_H:], bs1[0][None])
  U0, U1, deg = _edge_kernel_deg(row, col, A0, A1, B0, B1)
  for i in range(1, N_LAYERS):
    h, A0, A1, B0, B1 = _tc2(h, U0, U1, deg, Ws2[i - 1, :D_HALF],
                             Ws2[i - 1, D_HALF:], bs2[i - 1][None],
                             Ws1[i, :D_H], Ws1[i, D_H:], bs1[i][None])
    U0, U1 = _edge_kernel(row, col, A0, A1, B0, B1)
  hout, pooled = _tc3(h, U0, U1, deg, Ws2[N_LAYERS - 1, :D_HALF],
                      Ws2[N_LAYERS - 1, D_HALF:], bs2[N_LAYERS - 1][None],
                      W_out, b_out[None], bcol)
  return hout[:N_NODES], pooled


# speedup vs baseline: 6.9767x; 6.9767x over previous
import functools

import numpy as np

import jax
import jax.numpy as jnp
from jax import lax
from jax.experimental import pallas as pl
from jax.experimental.pallas import tpu as pltpu
from jax.experimental.pallas import tpu_sc as plsc

N_NODES = 10000
N_PAD = 10240
N_EDGES = 320000
D_IN = 128
D_H = 256
D_HALF = 128
D_OUT = 128
N_LAYERS = 4
N_GROUPS = 16

NC = 2
NS = 16
K = 80
EDGES_PER_SUB = N_EDGES // NS
CHUNKS = EDGES_PER_SUB // K
ROWS_PER_SUB = N_PAD // NS
LANES = 16

RB = 1024
GRID = N_PAD // RB

_BLK32 = np.arange(32).reshape(16, 2).T.ravel()
_SIGMA = np.concatenate([32 * f + _BLK32 for f in range(D_HALF // 32)])


def _build_edge_kernel():
  mesh = plsc.VectorSubcoreMesh(core_axis_name="c", subcore_axis_name="s",
                                num_cores=NC, num_subcores=NS)
  out_type = [
      jax.ShapeDtypeStruct((N_PAD, D_HALF), jnp.float32),
      jax.ShapeDtypeStruct((N_PAD, D_HALF), jnp.float32),
  ]
  scratch = [
      pltpu.VMEM((K,), jnp.int32),
      pltpu.VMEM((K,), jnp.int32),
      pltpu.VMEM((K,), jnp.int32),
      pltpu.VMEM((K,), jnp.int32),
      pltpu.VMEM((K, D_HALF), jnp.bfloat16),
      pltpu.VMEM((K, D_HALF), jnp.bfloat16),
      pltpu.VMEM((K, D_HALF), jnp.bfloat16),
      pltpu.VMEM((K, D_HALF), jnp.bfloat16),
      pltpu.VMEM((K, D_HALF), jnp.float32),
      pltpu.VMEM((K, D_HALF), jnp.float32),
      pltpu.VMEM((K,), jnp.int32),
      pltpu.VMEM((K,), jnp.int32),
      pltpu.VMEM_SHARED((N_PAD, D_HALF), jnp.float32),
      pltpu.SemaphoreType.DMA,
      pltpu.SemaphoreType.DMA,
      pltpu.SemaphoreType.DMA,
      pltpu.SemaphoreType.DMA,
      pltpu.SemaphoreType.DMA,
      pltpu.SemaphoreType.DMA,
      pltpu.SemaphoreType.DMA,
      pltpu.SemaphoreType.DMA,
  ]

  def body(row_hbm, col_hbm, a0_hbm, a1_hbm, b0_hbm, b1_hbm,
           u0_hbm, u1_hbm,
           rv0, cv0, rv1, cv1, av0, bv0, av1, bv1, ub0, ub1, rs0, rs1,
           u_sh, sa0, sb0, sa1, sb1, si0, si1, ss0, ss1):
    c = lax.axis_index("c")
    s = lax.axis_index("s")

    zero16 = jnp.zeros((LANES,), jnp.float32)

    def zrow(i, carry):
      for f in range(D_HALF // LANES):
        ub0[i, pl.ds(f * LANES, LANES)] = zero16
      return carry

    lax.fori_loop(0, K, zrow, 0)
    for j in range(ROWS_PER_SUB // K):
      pltpu.sync_copy(ub0, u_sh.at[pl.ds(s * ROWS_PER_SUB + j * K, K)])

    plsc.subcore_barrier()

    base = s * EDGES_PER_SUB

    def issue_idx(t, rv, cv, si):
      off = base + t * K
      pltpu.async_copy(row_hbm.at[pl.ds(off, K)], rv, si)
      pltpu.async_copy(col_hbm.at[pl.ds(off, K)], cv, si)

    def drain_idx(rv, cv, si):
      pltpu.make_async_copy(row_hbm.at[pl.ds(0, K)], rv, si).wait()
      pltpu.make_async_copy(col_hbm.at[pl.ds(0, K)], cv, si).wait()

    def issue_gather(rv, cv, av, bv, sa, sb):
      @pl.when(c == 0)
      def _():
        pltpu.async_copy(a0_hbm.at[rv], av, sa)
        pltpu.async_copy(b0_hbm.at[cv], bv, sb)

      @pl.when(c == 1)
      def _():
        pltpu.async_copy(a1_hbm.at[rv], av, sa)
        pltpu.async_copy(b1_hbm.at[cv], bv, sb)

    def drain_gather(av, bv, sa, sb):
      pltpu.make_async_copy(a0_hbm.at[pl.ds(0, K)], av, sa).wait()
      pltpu.make_async_copy(b0_hbm.at[pl.ds(0, K)], bv, sb).wait()

    himask = jnp.full((LANES,), -65536, jnp.int32)

    def process(rv, rs, av, bv, ub, ss):
      @plsc.parallel_loop(0, K, step=1, unroll=2)
      def _(i):
        for f in range(D_HALF // 32):
          sl = pl.ds(f * 32, 32)
          u = av[i, sl] + bv[i, sl]
          sg = u / (jnp.bfloat16(1.0) + jnp.exp(-u))
          w = plsc.bitcast(sg, jnp.int32)
          lo = plsc.bitcast(w << 16, jnp.float32)
          hi = plsc.bitcast(w & himask, jnp.float32)
          ub[i, pl.ds(f * 32, LANES)] = lo
          ub[i, pl.ds(f * 32 + LANES, LANES)] = hi

      for g in range(K // LANES):
        sl = pl.ds(g * LANES, LANES)
        rs[sl] = rv[sl]
      pltpu.async_copy(ub, u_sh.at[rs], ss, add=True)

    def drain_scatter(ub, rs, ss):
      pltpu.make_async_copy(ub, u_sh.at[rs], ss).wait()

    pltpu.sync_copy(row_hbm.at[pl.ds(base, K)], rv0)
    pltpu.sync_copy(col_hbm.at[pl.ds(base, K)], cv0)
    issue_gather(rv0, cv0, av0, bv0, sa0, sb0)
    issue_idx(1, rv1, cv1, si1)

    def body2(tt, carry):
      t0 = 2 * tt
      drain_idx(rv1, cv1, si1)
      issue_gather(rv1, cv1, av1, bv1, sa1, sb1)
      drain_gather(av0, bv0, sa0, sb0)

      @pl.when(tt > 0)
      def _():
        drain_scatter(ub0, rs0, ss0)

      process(rv0, rs0, av0, bv0, ub0, ss0)

      @pl.when(t0 + 2 < CHUNKS)
      def _():
        issue_idx(t0 + 2, rv0, cv0, si0)

      drain_gather(av1, bv1, sa1, sb1)

      @pl.when(tt > 0)
      def _():
        drain_scatter(ub1, rs1, ss1)

      process(rv1, rs1, av1, bv1, ub1, ss1)

      @pl.when(t0 + 2 < CHUNKS)
      def _():
        drain_idx(rv0, cv0, si0)
        issue_gather(rv0, cv0, av0, bv0, sa0, sb0)

      @pl.when(t0 + 3 < CHUNKS)
      def _():
        issue_idx(t0 + 3, rv1, cv1, si1)

      return carry

    lax.fori_loop(0, CHUNKS // 2, body2, 0)
    drain_scatter(ub0, rs0, ss0)
    drain_scatter(ub1, rs1, ss1)

    plsc.subcore_barrier()

    rsl = pl.ds(s * ROWS_PER_SUB, ROWS_PER_SUB)

    @pl.when(c == 0)
    def _():
      pltpu.sync_copy(u_sh.at[rsl], u0_hbm.at[rsl])

    @pl.when(c == 1)
    def _():
      pltpu.sync_copy(u_sh.at[rsl], u1_hbm.at[rsl])

  return pl.kernel(body, out_type=out_type, mesh=mesh,
                   scratch_types=scratch,
                   compiler_params=pltpu.CompilerParams(
                       use_tc_tiling_on_sc=False,
                       needs_layout_passes=False))


NW = NC * NS
EDGES_PER_W = N_EDGES // NW
DCHUNKS = EDGES_PER_W // K


def _build_deg_kernel():
  mesh = plsc.VectorSubcoreMesh(core_axis_name="c", subcore_axis_name="s",
                                num_cores=NC, num_subcores=NS)
  out_type = jax.ShapeDtypeStruct((NW * N_PAD,), jnp.float32)
  scratch = [
      pltpu.VMEM((K,), jnp.int32),
      pltpu.VMEM((K,), jnp.int32),
      pltpu.VMEM((N_PAD,), jnp.float32),
      pltpu.SemaphoreType.DMA,
      pltpu.SemaphoreType.DMA,
  ]

  def body(row_hbm, deg_hbm, rva, rvb, deg_v, sia, sib):
    c = lax.axis_index("c")
    s = lax.axis_index("s")
    wid = s * NC + c

    zero16 = jnp.zeros((LANES,), jnp.float32)
    one16 = jnp.ones((LANES,), jnp.float32)

    def zdeg(i, carry):
      deg_v[pl.ds(i * LANES, LANES)] = zero16
      return carry

    lax.fori_loop(0, N_PAD // LANES, zdeg, 0)

    base = wid * EDGES_PER_W

    def issue(t, rv, si):
      pltpu.async_copy(row_hbm.at[pl.ds(base + t * K, K)], rv, si)

    def drain(rv, si):
      pltpu.make_async_copy(row_hbm.at[pl.ds(0, K)], rv, si).wait()

    def accum(rv):
      for g in range(K // LANES):
        iv = rv[pl.ds(g * LANES, LANES)]
        plsc.addupdate_scatter(deg_v, [iv], one16)

    issue(0, rva, sia)
    issue(1, rvb, sib)

    def body2(tt, carry):
      t0 = 2 * tt
      drain(rva, sia)
      accum(rva)

      @pl.when(t0 + 2 < DCHUNKS)
      def _():
        issue(t0 + 2, rva, sia)

      @pl.when(t0 + 1 < DCHUNKS)
      def _():
        drain(rvb, sib)
        accum(rvb)

      @pl.when(t0 + 3 < DCHUNKS)
      def _():
        issue(t0 + 3, rvb, sib)

      return carry

    lax.fori_loop(0, (DCHUNKS + 1) // 2, body2, 0)

    pltpu.sync_copy(deg_v, deg_hbm.at[pl.ds(wid * N_PAD, N_PAD)])

  return pl.kernel(body, out_type=out_type, mesh=mesh,
                   scratch_types=scratch,
                   compiler_params=pltpu.CompilerParams(
                       use_tc_tiling_on_sc=False,
                       needs_layout_passes=False))


@functools.lru_cache(maxsize=None)
def _edge_kernel_cached():
  return _build_edge_kernel()


@functools.lru_cache(maxsize=None)
def _deg_kernel_cached():
  return _build_deg_kernel()


def _dot(a, b):
  return jnp.dot(a, b, preferred_element_type=jnp.float32)


def _tc1_body(x_ref, win_ref, bin_ref, w1a_ref, w1b_ref, bs1_ref,
              h_ref, a0_ref, a1_ref, b0_ref, b1_ref):
  hb = _dot(x_ref[...], win_ref[...]) + bin_ref[...]
  h_ref[...] = hb
  ab = (_dot(hb, w1a_ref[...]) + bs1_ref[...]).astype(jnp.bfloat16)
  a0_ref[...] = ab[:, :D_HALF]
  a1_ref[...] = ab[:, D_HALF:]
  bb = _dot(hb, w1b_ref[...]).astype(jnp.bfloat16)
  b0_ref[...] = bb[:, :D_HALF]
  b1_ref[...] = bb[:, D_HALF:]


def _agg(h_ref, u0_ref, u1_ref, deg_ref, w2t_ref, w2b_ref, bs2_ref):
  agg = _dot(u0_ref[...], w2t_ref[...]) + _dot(u1_ref[...], w2b_ref[...])
  agg = agg + deg_ref[...][:, 0:1] * bs2_ref[...]
  return h_ref[...] + agg


def _tc2_body(h_ref, u0_ref, u1_ref, deg_ref, w2t_ref, w2b_ref, bs2_ref,
              w1a_ref, w1b_ref, bs1_ref,
              hn_ref, a0_ref, a1_ref, b0_ref, b1_ref):
  hb = _agg(h_ref, u0_ref, u1_ref, deg_ref, w2t_ref, w2b_ref, bs2_ref)
  hn_ref[...] = hb
  ab = (_dot(hb, w1a_ref[...]) + bs1_ref[...]).astype(jnp.bfloat16)
  a0_ref[...] = ab[:, :D_HALF]
  a1_ref[...] = ab[:, D_HALF:]
  bb = _dot(hb, w1b_ref[...]).astype(jnp.bfloat16)
  b0_ref[...] = bb[:, :D_HALF]
  b1_ref[...] = bb[:, D_HALF:]


def _tc3_body(h_ref, u0_ref, u1_ref, deg_ref, w2t_ref, w2b_ref, bs2_ref,
              wout_ref, bout_ref, bcol_ref,
              hout_ref, pooled_ref, sums_ref, cnts_ref):
  pid = pl.program_id(0)
  hb = _agg(h_ref, u0_ref, u1_ref, deg_ref, w2t_ref, w2b_ref, bs2_ref)
  ho = _dot(hb, wout_ref[...]) + bout_ref[...]
  hout_ref[...] = ho

  bf = bcol_ref[...][:, 0:1]
  gids = lax.broadcasted_iota(jnp.int32, (1, N_GROUPS), 1)
  maskT = (bf == gids).astype(jnp.float32)
  dn = (((0,), (0,)), ((), ()))
  psums = lax.dot_general(maskT, ho, dn,
                          preferred_element_type=jnp.float32)
  pcnts = lax.dot_general(maskT, jnp.ones((RB, D_OUT), jnp.float32), dn,
                          preferred_element_type=jnp.float32)

  @pl.when(pid == 0)
  def _():
    sums_ref[...] = psums
    cnts_ref[...] = pcnts

  @pl.when(pid > 0)
  def _():
    sums_ref[...] += psums
    cnts_ref[...] += pcnts

  @pl.when(pid == GRID - 1)
  def _():
    cnt = cnts_ref[...]
    pooled_ref[...] = sums_ref[...] / cnt / jnp.sqrt(cnt)


def _row_spec(width):
  return pl.BlockSpec((RB, width), lambda i: (i, 0))


def _full_spec(shape):
  return pl.BlockSpec(shape, lambda i: tuple(0 for _ in shape))


_tc1 = pl.pallas_call(
    _tc1_body,
    grid=(GRID,),
    in_specs=[
        _row_spec(D_IN),
        _full_spec((D_IN, D_H)), _full_spec((1, D_H)),
        _full_spec((D_H, D_H)), _full_spec((D_H, D_H)), _full_spec((1, D_H)),
    ],
    out_specs=[
        _row_spec(D_H),
        _row_spec(D_HALF), _row_spec(D_HALF),
        _row_spec(D_HALF), _row_spec(D_HALF),
    ],
    out_shape=[
        jax.ShapeDtypeStruct((N_PAD, D_H), jnp.float32),
        jax.ShapeDtypeStruct((N_PAD, D_HALF), jnp.bfloat16),
        jax.ShapeDtypeStruct((N_PAD, D_HALF), jnp.bfloat16),
        jax.ShapeDtypeStruct((N_PAD, D_HALF), jnp.bfloat16),
        jax.ShapeDtypeStruct((N_PAD, D_HALF), jnp.bfloat16),
    ],
)

_tc2 = pl.pallas_call(
    _tc2_body,
    grid=(GRID,),
    in_specs=[
        _row_spec(D_H),
        _row_spec(D_HALF), _row_spec(D_HALF),
        _row_spec(LANES),
        _full_spec((D_HALF, D_H)), _full_spec((D_HALF, D_H)),
        _full_spec((1, D_H)),
        _full_spec((D_H, D_H)), _full_spec((D_H, D_H)), _full_spec((1, D_H)),
    ],
    out_specs=[
        _row_spec(D_H),
        _row_spec(D_HALF), _row_spec(D_HALF),
        _row_spec(D_HALF), _row_spec(D_HALF),
    ],
    out_shape=[
        jax.ShapeDtypeStruct((N_PAD, D_H), jnp.float32),
        jax.ShapeDtypeStruct((N_PAD, D_HALF), jnp.bfloat16),
        jax.ShapeDtypeStruct((N_PAD, D_HALF), jnp.bfloat16),
        jax.ShapeDtypeStruct((N_PAD, D_HALF), jnp.bfloat16),
        jax.ShapeDtypeStruct((N_PAD, D_HALF), jnp.bfloat16),
    ],
)

_tc3 = pl.pallas_call(
    _tc3_body,
    grid=(GRID,),
    in_specs=[
        _row_spec(D_H),
        _row_spec(D_HALF), _row_spec(D_HALF),
        _row_spec(LANES),
        _full_spec((D_HALF, D_H)), _full_spec((D_HALF, D_H)),
        _full_spec((1, D_H)),
        _full_spec((D_H, D_OUT)), _full_spec((1, D_OUT)),
        _row_spec(8),
    ],
    out_specs=[
        _row_spec(D_OUT),
        _full_spec((N_GROUPS, D_OUT)),
    ],
    out_shape=[
        jax.ShapeDtypeStruct((N_NODES, D_OUT), jnp.float32),
        jax.ShapeDtypeStruct((N_GROUPS, D_OUT), jnp.float32),
    ],
    scratch_shapes=[
        pltpu.VMEM((N_GROUPS, D_OUT), jnp.float32),
        pltpu.VMEM((N_GROUPS, D_OUT), jnp.float32),
    ],
)


@jax.jit
def kernel(x, edge_index, batch, W_in, b_in, Ws1, bs1, Ws2, bs2, W_out,
           b_out):
  row = edge_index[0]
  col = edge_index[1]
  x_pad = jnp.zeros((N_PAD, D_IN), jnp.float32).at[:N_NODES].set(x)
  bcol = jnp.full((N_PAD,), N_GROUPS, jnp.int32).at[:N_NODES].set(batch)
  bcol = jnp.broadcast_to(bcol[:, None], (N_PAD, 8))

  h, A0, A1, B0, B1 = _tc1(x_pad, W_in, b_in[None], Ws1[0, :D_H],
                           Ws1[0, D_H:], bs1[0][None])
  degP = _deg_kernel_cached()(row)
  deg = jnp.broadcast_to(
      degP.reshape(NW, N_PAD).sum(axis=0)[:, None], (N_PAD, LANES))
  U0, U1 = _edge_kernel_cached()(row, col, A0, A1, B0, B1)
  for i in range(1, N_LAYERS):
    h, A0, A1, B0, B1 = _tc2(h, U0, U1, deg, Ws2[i - 1, :D_HALF][_SIGMA],
                             Ws2[i - 1, D_HALF:][_SIGMA], bs2[i - 1][None],
                             Ws1[i, :D_H], Ws1[i, D_H:], bs1[i][None])
    U0, U1 = _edge_kernel_cached()(row, col, A0, A1, B0, B1)
  hout, pooled = _tc3(h, U0, U1, deg, Ws2[N_LAYERS - 1, :D_HALF][_SIGMA],
                      Ws2[N_LAYERS - 1, D_HALF:][_SIGMA],
                      bs2[N_LAYERS - 1][None], W_out, b_out[None], bcol)
  return hout, pooled
